# baseline (speedup 1.0000x reference)
import jax
import jax.numpy as jnp
from jax import lax
from jax.experimental import pallas as pl
from jax.experimental.pallas import tpu as pltpu
from jax.experimental.pallas import tpu_sc as plsc

VOCAB = 1000000
DIM = 64
PAD = 0

_NC = 2
_NS = 16
_L = 16
_NW = _NC * _NS

_B = 8192
_BPW = _B // _NW
_NB = 11
_TC = 128
_NG = _BPW // _L


def _sc_body(idx_hbm, lutT_hbm, out_hbm, idx_v, rows_v, *rest):
    bufs = rest[:_NB]
    sems = rest[_NB:]
    wid = lax.axis_index("s") * _NC + lax.axis_index("c")
    base = wid * _BPW

    for j in range(2):
        pltpu.sync_copy(idx_hbm.at[2 * wid + j],
                        idx_v.at[pl.ds(j * 128, 128)])

    onesf = jnp.ones((_L,), jnp.float32)
    zerosf = jnp.zeros((_L,), jnp.float32)
    dnums = lax.GatherDimensionNumbers(
        offset_dims=(), collapsed_slice_dims=(0,), start_index_map=(0,))

    def group_vecs(g):
        iv = idx_v[pl.ds(g * _L, _L)]
        return iv, (iv >> 7) << 7

    def fire(k, bc_s):
        return pltpu.async_copy(
            lutT_hbm.at[:, pl.ds(pl.multiple_of(bc_s, _TC), _TC)],
            bufs[k % _NB],
            sems[k % _NB],
        )

    dvecs = [lax.iota(jnp.int32, _L) + (m * _L) for m in range(DIM // _L)]
    handles = [None] * _BPW

    _, bc0 = group_vecs(0)
    for r in range(_NB):
        handles[r] = fire(r, bc0[r])

    for g in range(_NG):
        iv_g, bc_g = group_vecs(g)
        colrel = iv_g - bc_g
        scale = jnp.where(iv_g == PAD, zerosf, onesf)
        if g + 1 < _NG:
            _, bc_n = group_vecs(g + 1)
        for r in range(_L):
            k = g * _L + r
            handles[k].wait()
            bidx = jnp.full((_L, 1), r, jnp.int32)
            col_b = lax.gather(colrel, bidx, dnums, (1,),
                               mode=lax.GatherScatterMode.PROMISE_IN_BOUNDS)
            sc_b = lax.gather(scale, bidx, dnums, (1,),
                              mode=lax.GatherScatterMode.PROMISE_IN_BOUNDS)
            buf = bufs[k % _NB]
            for m in range(DIM // _L):
                val = plsc.load_gather(buf, [dvecs[m], col_b])
                rows_v[k, pl.ds(m * _L, _L)] = val * sc_b
            kf = k + _NB
            if kf < _BPW:
                bc_s = bc_g[r + _NB] if r + _NB < _L else bc_n[r + _NB - _L]
                handles[kf] = fire(kf, bc_s)

    pltpu.sync_copy(rows_v, out_hbm.at[pl.ds(base, _BPW)])


def _lookup(idx2d, lutT):
    mesh = plsc.VectorSubcoreMesh(core_axis_name="c", subcore_axis_name="s")
    return pl.kernel(
        _sc_body,
        out_type=jax.ShapeDtypeStruct((_B, DIM), jnp.float32),
        mesh=mesh,
        compiler_params=pltpu.CompilerParams(needs_layout_passes=False),
        scratch_types=[
            pltpu.VMEM((_BPW,), jnp.int32),
            pltpu.VMEM((_BPW, DIM), jnp.float32),
        ] + [pltpu.VMEM((DIM, _TC), jnp.float32)] * _NB
          + [pltpu.SemaphoreType.DMA] * _NB,
    )(idx2d, lutT)


def kernel(src_input, word_lut):
    seq, batch, _ = src_input.shape
    idx2d = src_input[:, :, 0].reshape(_B // 128, 128)
    out = _lookup(idx2d, word_lut.T)
    return out.reshape(seq, batch, DIM)


# speedup vs baseline: 1.0187x; 1.0187x over previous
import jax
import jax.numpy as jnp
from jax import lax
from jax.experimental import pallas as pl
from jax.experimental.pallas import tpu as pltpu
from jax.experimental.pallas import tpu_sc as plsc

VOCAB = 1000000
DIM = 64
PAD = 0

_NC = 2
_NS = 16
_L = 16
_NW = _NC * _NS

_B = 8192
_BPW = _B // _NW
_NB = 10
_TC = 128
_NG = _BPW // _L


def _sc_body(idx_hbm, lutT_hbm, out_hbm, idx_v, rows_v, *rest):
    bufs = rest[:_NB]
    sems = rest[_NB:]
    wid = lax.axis_index("s") * _NC + lax.axis_index("c")
    base = wid * _BPW

    for j in range(2):
        pltpu.sync_copy(idx_hbm.at[2 * wid + j],
                        idx_v.at[pl.ds(j * 128, 128)])

    onesf = jnp.ones((_L,), jnp.float32)
    zerosf = jnp.zeros((_L,), jnp.float32)
    dnums = lax.GatherDimensionNumbers(
        offset_dims=(), collapsed_slice_dims=(0,), start_index_map=(0,))

    def group_vecs(g):
        iv = idx_v[pl.ds(g * _L, _L)]
        return iv, (iv >> 7) << 7

    def fire(k, bc_s):
        return pltpu.async_copy(
            lutT_hbm.at[:, pl.ds(pl.multiple_of(bc_s, _TC), _TC)],
            bufs[k % _NB],
            sems[k % _NB],
        )

    dvecs = [lax.iota(jnp.int32, _L) + (m * _L) for m in range(DIM // _L)]
    handles = [None] * _BPW

    _, bc0 = group_vecs(0)
    for r in range(_NB):
        handles[r] = fire(r, bc0[r])

    for g in range(_NG):
        iv_g, bc_g = group_vecs(g)
        colrel = iv_g - bc_g
        scale = jnp.where(iv_g == PAD, zerosf, onesf)
        if g + 1 < _NG:
            _, bc_n = group_vecs(g + 1)
        for r in range(_L):
            k = g * _L + r
            handles[k].wait()
            bidx = jnp.full((_L, 1), r, jnp.int32)
            col_b = lax.gather(colrel, bidx, dnums, (1,),
                               mode=lax.GatherScatterMode.PROMISE_IN_BOUNDS)
            sc_b = lax.gather(scale, bidx, dnums, (1,),
                              mode=lax.GatherScatterMode.PROMISE_IN_BOUNDS)
            buf = bufs[k % _NB]
            for m in range(DIM // _L):
                val = plsc.load_gather(buf, [dvecs[m], col_b])
                rows_v[k, pl.ds(m * _L, _L)] = val * sc_b
            kf = k + _NB
            if kf < _BPW:
                bc_s = bc_g[r + _NB] if r + _NB < _L else bc_n[r + _NB - _L]
                handles[kf] = fire(kf, bc_s)

    pltpu.sync_copy(rows_v, out_hbm.at[pl.ds(base, _BPW)])


def _lookup(idx2d, lutT):
    mesh = plsc.VectorSubcoreMesh(core_axis_name="c", subcore_axis_name="s")
    return pl.kernel(
        _sc_body,
        out_type=jax.ShapeDtypeStruct((_B, DIM), jnp.float32),
        mesh=mesh,
        compiler_params=pltpu.CompilerParams(needs_layout_passes=False),
        scratch_types=[
            pltpu.VMEM((_BPW,), jnp.int32),
            pltpu.VMEM((_BPW, DIM), jnp.float32),
        ] + [pltpu.VMEM((DIM, _TC), jnp.float32)] * _NB
          + [pltpu.SemaphoreType.DMA] * _NB,
    )(idx2d, lutT)


def kernel(src_input, word_lut):
    seq, batch, _ = src_input.shape
    idx2d = src_input[:, :, 0].reshape(_B // 128, 128)
    out = _lookup(idx2d, word_lut.T)
    return out.reshape(seq, batch, DIM)
